# Initial kernel scaffold; baseline (speedup 1.0000x reference)
import jax
import jax.numpy as jnp
from jax.experimental import pallas as pl


def kernel(input, indices):
    raise NotImplementedError("write your pallas kernel here")


# speedup vs baseline: 3.8309x; 3.8309x over previous
import functools

import jax
import jax.numpy as jnp
from jax import lax
from jax.experimental import pallas as pl
from jax.experimental.pallas import tpu as pltpu
from jax.experimental.pallas import tpu_sc as plsc

N, C, H_IN, H_OUT = 16, 768, 2048, 4096
ROWS = N * C
NUM_WORKERS = 32
ROWS_PER_W = ROWS // NUM_WORKERS
K = 8
CHUNKS = ROWS_PER_W // K
L = 16
VREGS_IN = K * H_IN // L
VREGS_OUT = K * H_OUT // L


def _sc_scatter(key_flat, val_flat):
    mesh = plsc.VectorSubcoreMesh(core_axis_name="c", subcore_axis_name="s")

    @functools.partial(
        pl.kernel,
        mesh=mesh,
        out_type=jax.ShapeDtypeStruct((ROWS * H_OUT,), jnp.float32),
        scratch_types=[
            pltpu.VMEM((K * H_IN,), jnp.int32),
            pltpu.VMEM((K * H_IN,), jnp.float32),
            pltpu.VMEM((K * H_OUT,), jnp.float32),
        ],
        compiler_params=pltpu.CompilerParams(needs_layout_passes=False),
    )
    def body(key_hbm, val_hbm, out_hbm, key_v, val_v, out_v):
        wid = lax.axis_index("s") * 2 + lax.axis_index("c")
        base = wid * ROWS_PER_W

        def chunk_loop(ci, carry):
            row0 = base + ci * K
            pltpu.sync_copy(key_hbm.at[pl.ds(row0 * H_IN, K * H_IN)], key_v)
            pltpu.sync_copy(val_hbm.at[pl.ds(row0 * H_IN, K * H_IN)], val_v)

            zero = jnp.zeros((L,), jnp.float32)

            def zfill(j, c):
                out_v[pl.ds(j * L, L)] = zero
                return c

            lax.fori_loop(0, VREGS_OUT, zfill, 0)

            base_key = row0 * H_OUT

            def scat(j, c):
                iv = key_v[pl.ds(j * L, L)] - base_key
                vv = val_v[pl.ds(j * L, L)]
                plsc.store_scatter(out_v, [iv], vv)
                return c

            lax.fori_loop(0, VREGS_IN, scat, 0)

            pltpu.sync_copy(out_v, out_hbm.at[pl.ds(row0 * H_OUT, K * H_OUT)])
            return carry

        lax.fori_loop(0, CHUNKS, chunk_loop, 0)

    return body(key_flat, val_flat)


def kernel(input, indices):
    idx = indices.astype(jnp.int32)
    key = (idx
           + (jnp.arange(C, dtype=jnp.int32) * H_OUT)[None, :, None]
           + (jnp.arange(N, dtype=jnp.int32) * (C * H_OUT))[:, None, None])
    sk, sv = lax.sort_key_val(key.reshape(-1), input.reshape(-1), is_stable=False)
    out = _sc_scatter(sk, sv)
    return out.reshape(N, C, H_OUT)
